# Initial kernel scaffold; baseline (speedup 1.0000x reference)
import jax
import jax.numpy as jnp
from jax.experimental import pallas as pl


def kernel(node_type, edge_index, edge_type, emb, W, W_root, b):
    raise NotImplementedError("write your pallas kernel here")


# speedup vs baseline: 14.1864x; 14.1864x over previous
import functools

import jax
import jax.numpy as jnp
from jax import lax
from jax.experimental import pallas as pl
from jax.experimental.pallas import tpu as pltpu
from jax.experimental.pallas import tpu_sc as plsc

N, E, D, R, T = 10000, 320000, 128, 8, 16

NC, NS = 2, 16
NW = NC * NS
CHUNK = 128
EPW = -(-E // NW)
CPW = -(-EPW // CHUNK)
EPAD = NW * CPW * CHUNK

NP = 10240
BN = 512
NBLK = NP // BN
ACC = NP
RPT = ACC // NS
ZCH = RPT // CHUNK


def _sc_agg_body(y_hbm, gi_hbm, di_hbm, z_hbm, out_hbm,
                 acc, gi_v, di_v, rows_v, sem):
    c = lax.axis_index("c")
    s = lax.axis_index("s")
    w = c * NS + s
    pltpu.sync_copy(gi_hbm.at[w], gi_v)
    pltpu.sync_copy(di_hbm.at[w], di_v)
    pltpu.sync_copy(z_hbm, rows_v)
    for k in range(ZCH):
        pltpu.sync_copy(rows_v, acc.at[pl.ds(s * RPT + k * CHUNK, CHUNK)])
    plsc.subcore_barrier()

    def chunk(j, carry):
        pltpu.async_copy(y_hbm.at[gi_v.at[j]], rows_v, sem).wait()
        pltpu.sync_copy(rows_v, acc.at[di_v.at[j]], add=True)
        return carry

    lax.fori_loop(0, CPW, chunk, 0)
    plsc.subcore_barrier()
    for k in range(ZCH):
        sl = pl.ds(s * RPT + k * CHUNK, CHUNK)
        pltpu.sync_copy(acc.at[sl], rows_v)
        pltpu.sync_copy(rows_v, out_hbm.at[c].at[sl])


_sc_agg = pl.kernel(
    _sc_agg_body,
    out_type=jax.ShapeDtypeStruct((NC, ACC, D), jnp.float32),
    mesh=plsc.VectorSubcoreMesh(core_axis_name="c", subcore_axis_name="s"),
    scratch_types=[
        pltpu.VMEM_SHARED((ACC, D), jnp.float32),
        pltpu.VMEM((CPW, CHUNK), jnp.int32),
        pltpu.VMEM((CPW, CHUNK), jnp.int32),
        pltpu.VMEM((CHUNK, D), jnp.float32),
        pltpu.SemaphoreType.DMA,
    ],
)


def _tc_first_body(nt_ref, emb_ref, wc_ref, wr_ref, b_ref, y_ref, root_ref):
    nt = nt_ref[0]
    oh = (jnp.broadcast_to(nt, (T, BN))
          == lax.broadcasted_iota(jnp.int32, (T, BN), 0)).astype(jnp.float32)
    x = lax.dot_general(oh, emb_ref[...], (((0,), (0,)), ((), ())),
                        preferred_element_type=jnp.float32)
    y_ref[...] = jnp.dot(x, wc_ref[...], preferred_element_type=jnp.float32)
    root_ref[...] = (jnp.dot(x, wr_ref[...], preferred_element_type=jnp.float32)
                     + b_ref[...])


def _tc_mid_body(a0_ref, a1_ref, rp_ref, wc_ref, wr_ref, b_ref, y_ref, root_ref):
    z = a0_ref[...] + a1_ref[...] + rp_ref[...]
    x = jnp.where(z >= 0, z, 0.01 * z)
    y_ref[...] = jnp.dot(x, wc_ref[...], preferred_element_type=jnp.float32)
    root_ref[...] = (jnp.dot(x, wr_ref[...], preferred_element_type=jnp.float32)
                     + b_ref[...])


def _tc_final_body(a0_ref, a1_ref, rp_ref, x_ref):
    z = a0_ref[...] + a1_ref[...] + rp_ref[...]
    x_ref[...] = jnp.where(z >= 0, z, 0.01 * z)


_full = lambda shape: pl.BlockSpec(shape, lambda i: tuple(0 for _ in shape))
_rows = lambda shape: pl.BlockSpec(shape, lambda i: (i,) + tuple(0 for _ in shape[1:]))

_tc_first = pl.pallas_call(
    _tc_first_body,
    grid=(NBLK,),
    in_specs=[
        pl.BlockSpec((1, 1, BN), lambda i: (i, 0, 0)),
        _full((T, D)), _full((D, R * D)), _full((D, D)), _full((1, D)),
    ],
    out_specs=[_rows((BN, R * D)), _rows((BN, D))],
    out_shape=[jax.ShapeDtypeStruct((NP, R * D), jnp.float32),
               jax.ShapeDtypeStruct((NP, D), jnp.float32)],
)

_tc_mid = pl.pallas_call(
    _tc_mid_body,
    grid=(NBLK,),
    in_specs=[
        _rows((BN, D)), _rows((BN, D)), _rows((BN, D)),
        _full((D, R * D)), _full((D, D)), _full((1, D)),
    ],
    out_specs=[_rows((BN, R * D)), _rows((BN, D))],
    out_shape=[jax.ShapeDtypeStruct((NP, R * D), jnp.float32),
               jax.ShapeDtypeStruct((NP, D), jnp.float32)],
)

_tc_final = pl.pallas_call(
    _tc_final_body,
    grid=(NBLK,),
    in_specs=[_rows((BN, D)), _rows((BN, D)), _rows((BN, D))],
    out_specs=_rows((BN, D)),
    out_shape=jax.ShapeDtypeStruct((NP, D), jnp.float32),
)


def kernel(node_type, edge_index, edge_type, emb, W, W_root, b):
    L = W.shape[0]
    src = edge_index[0].astype(jnp.int32)
    dst = edge_index[1].astype(jnp.int32)
    et = edge_type.astype(jnp.int32)
    gi = jnp.concatenate([src * R + et, jnp.zeros((EPAD - E,), jnp.int32)])
    gi = gi.reshape(NW, CPW, CHUNK)
    di = jnp.concatenate([dst, jnp.full((EPAD - E,), N, jnp.int32)])
    di = di.reshape(NW, CPW, CHUNK)
    zeros_blk = jnp.zeros((CHUNK, D), jnp.float32)
    nt3 = jnp.concatenate([node_type.astype(jnp.int32),
                           jnp.zeros((NP - N,), jnp.int32)]).reshape(NBLK, 1, BN)
    Wcat = W.astype(jnp.float32).transpose(0, 2, 1, 3).reshape(L, D, R * D)
    Wr = W_root.astype(jnp.float32)
    brow = b.astype(jnp.float32).reshape(L, 1, D)

    y, root = _tc_first(nt3, emb.astype(jnp.float32), Wcat[0], Wr[0], brow[0])
    for l in range(L):
        agg = _sc_agg(y.reshape(NP * R, D), gi, di, zeros_blk)
        if l + 1 < L:
            y, root = _tc_mid(agg[0], agg[1], root,
                              Wcat[l + 1], Wr[l + 1], brow[l + 1])
        else:
            xf = _tc_final(agg[0], agg[1], root)
    return xf[:N]
